# baseline (speedup 1.0000x reference)
import functools

import jax
import jax.numpy as jnp
from jax import lax
from jax.experimental import pallas as pl
from jax.experimental.pallas import tpu as pltpu
from jax.experimental.pallas import tpu_sc as plsc

N = 10000
E = 320000
NC, NS = 2, 16
NW = NC * NS
EW = E // NW
CH = 125
NCHUNK = EW // CH
FS = 64
RPT = N // NS

_mesh = plsc.VectorSubcoreMesh(
    core_axis_name="c", subcore_axis_name="s", num_cores=NC, num_subcores=NS)


def _deg_kernel(col_hbm, out_hbm, cidx, ones_v, zb16, acc, sem):
    c = lax.axis_index("c")
    s = lax.axis_index("s")
    w = c * NS + s

    def zb(i, carry):
        ones_v[i, pl.ds(0, 16)] = jnp.ones((16,), jnp.float32)
        zb16[i, pl.ds(0, 16)] = jnp.zeros((16,), jnp.float32)
        return carry
    lax.fori_loop(0, CH, zb, 0)
    for t in range(RPT // CH):
        pltpu.sync_copy(zb16, acc.at[pl.ds(s * RPT + t * CH, CH)])
    pltpu.sync_copy(col_hbm.at[w], cidx)
    plsc.subcore_barrier()

    def body(i, carry):
        for k in range(8):
            pltpu.async_copy(ones_v, acc.at[cidx.at[8 * i + k]], sem, add=True)
        for k in range(8):
            pltpu.make_async_copy(ones_v, acc.at[cidx.at[0]], sem).wait()
        return carry

    lax.fori_loop(0, NCHUNK // 8, body, 0)
    plsc.subcore_barrier()
    pltpu.sync_copy(acc.at[pl.ds(s * RPT, RPT)], out_hbm.at[w])


_deg_kernel = functools.partial(
    pl.kernel,
    out_type=jax.ShapeDtypeStruct((NW, RPT, 16), jnp.float32),
    mesh=_mesh,
    compiler_params=pltpu.CompilerParams(use_tc_tiling_on_sc=False),
    scratch_types=[
        pltpu.VMEM((NCHUNK, CH), jnp.int32),
        pltpu.VMEM((CH, 16), jnp.float32),
        pltpu.VMEM((CH, 16), jnp.float32),
        pltpu.VMEM_SHARED((N, 16), jnp.float32),
        pltpu.SemaphoreType.DMA,
    ],
)(_deg_kernel)


def _make_propagate(m):
    F = FS
    NB = NCHUNK // 4

    @functools.partial(
        pl.kernel,
        out_type=tuple(jax.ShapeDtypeStruct((NW, RPT, F), jnp.float32)
                       for _ in range(m)),
        mesh=_mesh,
        compiler_params=pltpu.CompilerParams(use_tc_tiling_on_sc=False),
        scratch_types=[
            pltpu.VMEM((NCHUNK, CH), jnp.int32),
            pltpu.VMEM((NCHUNK, CH), jnp.int32),
            pltpu.VMEM((CH, F), jnp.float32),
            pltpu.VMEM((CH, F), jnp.float32),
            pltpu.VMEM((CH, F), jnp.float32),
            pltpu.VMEM((CH, F), jnp.float32),
            pltpu.VMEM((CH, F), jnp.float32),
            pltpu.VMEM_SHARED((N, F), jnp.float32),
        ] + [pltpu.SemaphoreType.DMA] * 8,
    )
    def _prop(*refs):
        gs_hbm = refs[:m]
        row_hbm, col_hbm = refs[m], refs[m + 1]
        outs = refs[m + 2:2 * m + 2]
        ridx, cidx, b0, b1, b2, b3, zbuf, acc = refs[2 * m + 2:2 * m + 10]
        sems = refs[2 * m + 10:]
        gsm, ssm = sems[:4], sems[4:]
        bufs = [b0, b1, b2, b3]
        c = lax.axis_index("c")
        s = lax.axis_index("s")
        w = c * NS + s
        base_r = s * RPT

        def zb(i, carry):
            for k in range(F // 16):
                zbuf[i, pl.ds(k * 16, 16)] = jnp.zeros((16,), jnp.float32)
            return carry
        lax.fori_loop(0, CH, zb, 0)
        for t in range(RPT // CH):
            pltpu.sync_copy(zbuf, acc.at[pl.ds(base_r + t * CH, CH)])
        pltpu.sync_copy(row_hbm.at[w], ridx)
        pltpu.sync_copy(col_hbm.at[w], cidx)
        for k in range(4):
            pltpu.async_copy(gs_hbm[0].at[ridx.at[k]], bufs[k], gsm[k])
        plsc.subcore_barrier()

        for si in range(m):
            g_hbm = gs_hbm[si]

            def body(i, carry, g_hbm=g_hbm):
                j0 = 4 * i
                descs = []
                for k in range(4):
                    pltpu.make_async_copy(
                        g_hbm.at[ridx.at[0]], bufs[k], gsm[k]).wait()
                    descs.append(pltpu.async_copy(
                        bufs[k], acc.at[cidx.at[j0 + k]], ssm[k], add=True))
                for k in range(4):
                    descs[k].wait()
                    pltpu.async_copy(
                        g_hbm.at[ridx.at[j0 + 4 + k]], bufs[k], gsm[k])
                return carry

            lax.fori_loop(0, NB - 1, body, 0)
            j0 = 4 * (NB - 1)
            descs = []
            for k in range(4):
                pltpu.make_async_copy(
                    g_hbm.at[ridx.at[0]], bufs[k], gsm[k]).wait()
                descs.append(pltpu.async_copy(
                    bufs[k], acc.at[cidx.at[j0 + k]], ssm[k], add=True))
            for k in range(4):
                descs[k].wait()
            if si < m - 1:
                for k in range(4):
                    pltpu.async_copy(
                        gs_hbm[si + 1].at[ridx.at[k]], bufs[k], gsm[k])
            plsc.subcore_barrier()
            pltpu.sync_copy(acc.at[pl.ds(base_r, RPT)], outs[si].at[w])
            if si < m - 1:
                for t in range(RPT // CH):
                    pltpu.sync_copy(
                        zbuf, acc.at[pl.ds(base_r + t * CH, CH)])
                plsc.subcore_barrier()

    return _prop


_prop1 = _make_propagate(1)
_prop2 = _make_propagate(2)
_prop4 = _make_propagate(4)


def _dis_body(p_ref, o_ref):
    o_ref[...] = lax.rsqrt(1.0 + p_ref[0:1] + p_ref[1:2])


def _dis_tc(partials):
    return pl.pallas_call(
        _dis_body,
        out_shape=jax.ShapeDtypeStruct((1, N), jnp.float32),
    )(partials)


def _mm_body(widths, col0, dis_ref, *refs):
    xs = refs[:len(widths)]
    w_ref = refs[len(widths)]
    o_ref = refs[len(widths) + 1]
    acc = None
    off = 0
    for x_ref, wd in zip(xs, widths):
        part = jnp.dot(dis_ref[...] * x_ref[...],
                       w_ref[off:off + wd, col0:col0 + FS],
                       preferred_element_type=jnp.float32)
        acc = part if acc is None else acc + part
        off += wd
    o_ref[...] = acc


def _mm_tc(dis, xs, W, col0, rb=2000):
    widths = tuple(xx.shape[1] for xx in xs)
    return pl.pallas_call(
        functools.partial(_mm_body, widths, col0),
        grid=(N // rb,),
        in_specs=[pl.BlockSpec((rb, 1), lambda i: (i, 0))]
        + [pl.BlockSpec((rb, wd), lambda i: (i, 0)) for wd in widths]
        + [pl.BlockSpec(W.shape, lambda i: (0, 0))],
        out_specs=pl.BlockSpec((rb, FS), lambda i: (i, 0)),
        out_shape=jax.ShapeDtypeStruct((N, FS), jnp.float32),
    )(dis, *xs, W)


def _comb_body(relu, s_ref, g_ref, dis_ref, b_ref, o_ref):
    v = dis_ref[...] * (s_ref[0] + s_ref[1] + g_ref[...]) + b_ref[...]
    if relu:
        v = jnp.maximum(v, 0.0)
    o_ref[...] = v


def _comb_tc(S, g, dis, b, relu, rb=2000):
    return pl.pallas_call(
        functools.partial(_comb_body, relu),
        grid=(N // rb,),
        in_specs=[
            pl.BlockSpec((NC, rb, FS), lambda i: (0, i, 0)),
            pl.BlockSpec((rb, FS), lambda i: (i, 0)),
            pl.BlockSpec((rb, 1), lambda i: (i, 0)),
            pl.BlockSpec((1, FS), lambda i: (0, 0)),
        ],
        out_specs=pl.BlockSpec((rb, FS), lambda i: (i, 0)),
        out_shape=jax.ShapeDtypeStruct((N, FS), jnp.float32),
    )(S, g, dis, b)


def _pack_body(*refs):
    o_ref = refs[-1]
    o_ref[...] = jnp.concatenate([r[...] for r in refs[:-1]], axis=1)


def _pack_tc(slices, rb=2000):
    return pl.pallas_call(
        _pack_body,
        grid=(N // rb,),
        in_specs=[pl.BlockSpec((rb, FS), lambda i: (i, 0)) for _ in slices],
        out_specs=pl.BlockSpec((rb, 7 * FS), lambda i: (i, 0)),
        out_shape=jax.ShapeDtypeStruct((N, 7 * FS), jnp.float32),
    )(*slices)


def kernel(x, edge_index, emb, W1, b1, W2, b2, W3, b3):
    ei = edge_index.astype(jnp.int32)
    row = ei[0].reshape(NW, NCHUNK, CH)
    col = ei[1].reshape(NW, NCHUNK, CH)
    x_full = jnp.concatenate([x, emb], axis=0)

    deg_parts = _deg_kernel(col).reshape(NC, N, 16)[:, :, 0]
    dis = _dis_tc(deg_parts).reshape(N, 1)

    def layer(xs_in, W, b, relu, propm):
        m = W.shape[1] // FS
        gs = [_mm_tc(dis, xs_in, W, k * FS) for k in range(m)]
        Ss = propm(*gs, row, col)
        if m == 1:
            Ss = (Ss,) if not isinstance(Ss, (tuple, list)) else Ss
        return [_comb_tc(Ss[k].reshape(NC, N, FS), gs[k], dis,
                         b[k * FS:(k + 1) * FS].reshape(1, -1), relu)
                for k in range(m)]

    x1s = layer([x_full], W1, b1, True, _prop2)
    x2s = layer(x1s, W2, b2, True, _prop4)
    x3s = layer(x2s, W3, b3, False, _prop1)

    return _pack_tc(x3s + x2s + x1s)
